# baseline (speedup 1.0000x reference)
import jax
import jax.numpy as jnp
from jax.experimental import pallas as pl
from jax.experimental.pallas import tpu as pltpu

_B, _N, _M = 8, 512, 2048
_SCALE_DIV = 1.2
_ITERS = 8
_TRIM = 5.0
_HUBER = 1.0
_SWEEPS = 5


def _bf(x):
    return x.astype(jnp.bfloat16).astype(jnp.float32)


def _trunc16(x):
    u = jax.lax.bitcast_convert_type(x, jnp.uint32)
    return jax.lax.bitcast_convert_type(u & jnp.uint32(0xFFFF0000), jnp.float32)


def _dot(a, b, ca, cb):
    return jax.lax.dot_general(
        a, b, (((ca,), (cb,)), ((), ())),
        preferred_element_type=jnp.float32)


def _icp_body(scan_ref, map_ref, mapT_ref, tinit_ref, p_ref, out_ref):
    scale = jnp.maximum(p_ref[0:1, 0:1], 0.0)

    sbs = []
    mTbs, msqs, msplits = [], [], []
    for b in range(_B):
        sx = (scan_ref[b, :, 0:1] / _SCALE_DIV) * scale
        sy = (scan_ref[b, :, 1:2] / _SCALE_DIV) * scale
        sz = (scan_ref[b, :, 2:3] / _SCALE_DIV) * scale
        sbs.append((_bf(sx), _bf(sy), _bf(sz)))
        mx = mapT_ref[b, 0:1, :]
        my = mapT_ref[b, 1:2, :]
        mz = mapT_ref[b, 2:3, :]
        mTbs.append(mapT_ref[b].astype(jnp.bfloat16))
        msqs.append(mx * mx + my * my + mz * mz)
        mp_mat = map_ref[b]
        m1f = _trunc16(mp_mat)
        r1 = mp_mat - m1f
        m2f = _trunc16(r1)
        r2 = r1 - m2f
        msplits.append((m1f.astype(jnp.bfloat16), m2f.astype(jnp.bfloat16),
                        r2.astype(jnp.bfloat16)))
    iota = jax.lax.broadcasted_iota(jnp.int32, (_N, _M), 1).astype(jnp.float32)

    def tcol(i, j):
        return jnp.concatenate(
            [tinit_ref[b, i:i + 1, j:j + 1] for b in range(_B)], axis=1)

    R0 = [[tcol(i, j) for j in range(3)] for i in range(3)]
    t0 = [tcol(i, 3) for i in range(3)]

    def body(_, carry):
        (r00, r01, r02, r10, r11, r12, r20, r21, r22, t0_, t1_, t2_) = carry
        R = [[r00, r01, r02], [r10, r11, r12], [r20, r21, r22]]
        t = [t0_, t1_, t2_]
        Rb = [[_bf(R[i][j]) for j in range(3)] for i in range(3)]

        sums = []
        for b in range(_B):
            sxb, syb, szb = sbs[b]

            def lane(v):
                return v[0:1, b:b + 1]

            px = (sxb * lane(Rb[0][0]) + syb * lane(Rb[0][1])) \
                + szb * lane(Rb[0][2]) + lane(t[0])
            py = (sxb * lane(Rb[1][0]) + syb * lane(Rb[1][1])) \
                + szb * lane(Rb[1][2]) + lane(t[1])
            pz = (sxb * lane(Rb[2][0]) + syb * lane(Rb[2][1])) \
                + szb * lane(Rb[2][2]) + lane(t[2])
            p_sq = px * px + py * py + pz * pz
            ptsb = jnp.concatenate([px, py, pz], axis=1).astype(jnp.bfloat16)

            cross = _dot(ptsb, mTbs[b], 1, 0)
            d2 = (p_sq + msqs[b]) - 2.0 * cross
            d2min = jnp.min(d2, axis=1, keepdims=True)
            hit = d2 == d2min
            idx = jnp.min(jnp.where(hit, iota, float(_M)), axis=1,
                          keepdims=True)
            oneb = (iota == idx).astype(jnp.bfloat16)
            m1b, m2b, m3b = msplits[b]
            nn = (_dot(oneb, m1b, 1, 0) + _dot(oneb, m2b, 1, 0)) \
                + _dot(oneb, m3b, 1, 0)
            nx = nn[:, 0:1]
            ny = nn[:, 1:2]
            nz = nn[:, 2:3]

            rx = px - nx
            ry = py - ny
            rz = pz - nz
            dist = jnp.sqrt(rx * rx + ry * ry + rz * rz + 1e-12)
            w_trim = (dist < _TRIM).astype(jnp.float32)
            w_hub = jnp.where(dist > _HUBER, _HUBER / dist, 1.0)
            w = w_trim * w_hub

            def rsum(v):
                return jnp.sum(v, axis=0, keepdims=True)

            sw_b = rsum(w) + 1e-9
            mu_p_b = [rsum(w * px) / sw_b, rsum(w * py) / sw_b,
                      rsum(w * pz) / sw_b]
            mu_q_b = [rsum(w * nx) / sw_b, rsum(w * ny) / sw_b,
                      rsum(w * nz) / sw_b]
            pc = [px - mu_p_b[0], py - mu_p_b[1], pz - mu_p_b[2]]
            qc = [nx - mu_q_b[0], ny - mu_q_b[1], nz - mu_q_b[2]]
            wpcm = jnp.concatenate([w * pc[0], w * pc[1], w * pc[2]],
                                   axis=1).astype(jnp.bfloat16)
            qcm = jnp.concatenate([qc[0], qc[1], qc[2]],
                                  axis=1).astype(jnp.bfloat16)
            Hm = _dot(wpcm, qcm, 0, 0)
            H_b = [[Hm[i:i + 1, j:j + 1] for j in range(3)]
                   for i in range(3)]
            sums.append((mu_p_b, mu_q_b, H_b))

        def pack(pick):
            return jnp.concatenate([pick(sums[b]) for b in range(_B)], axis=1)

        mu_p = [pack(lambda s, i=i: s[0][i]) for i in range(3)]
        mu_q = [pack(lambda s, i=i: s[1][i]) for i in range(3)]
        H = [[pack(lambda s, i=i, j=j: s[2][i][j]) for j in range(3)]
             for i in range(3)]

        def ata(i, j):
            return H[0][i] * H[0][j] + H[1][i] * H[1][j] + H[2][i] * H[2][j]

        a = [[ata(i, j) for j in range(3)] for i in range(3)]
        V = [[jnp.full((1, _B), 1.0 if i == j else 0.0, jnp.float32)
              for j in range(3)] for i in range(3)]

        for _s in range(_SWEEPS):
            for (p, q) in ((0, 1), (0, 2), (1, 2)):
                r = 3 - p - q
                app, aqq, apq = a[p][p], a[q][q], a[p][q]
                tiny = jnp.abs(apq) < 1e-37
                apq_safe = jnp.where(tiny, 1.0, apq)
                tau = (aqq - app) * 0.5 / apq_safe
                sgn = jnp.where(tau >= 0.0, 1.0, -1.0)
                tt = sgn / (jnp.abs(tau) + jnp.sqrt(1.0 + tau * tau))
                c = 1.0 / jnp.sqrt(1.0 + tt * tt)
                s = tt * c
                c = jnp.where(tiny, 1.0, c)
                s = jnp.where(tiny, 0.0, s)
                new_pp = c * c * app - 2.0 * s * c * apq + s * s * aqq
                new_qq = s * s * app + 2.0 * s * c * apq + c * c * aqq
                apr, aqr = a[p][r], a[q][r]
                new_pr = c * apr - s * aqr
                new_qr = s * apr + c * aqr
                a[p][p] = new_pp
                a[q][q] = new_qq
                a[p][q] = jnp.zeros((1, _B), jnp.float32)
                a[q][p] = a[p][q]
                a[p][r] = new_pr
                a[r][p] = new_pr
                a[q][r] = new_qr
                a[r][q] = new_qr
                for i in range(3):
                    vip, viq = V[i][p], V[i][q]
                    V[i][p] = c * vip - s * viq
                    V[i][q] = s * vip + c * viq

        eig = [a[0][0], a[1][1], a[2][2]]
        detH = (H[0][0] * (H[1][1] * H[2][2] - H[1][2] * H[2][1])
                - H[0][1] * (H[1][0] * H[2][2] - H[1][2] * H[2][0])
                + H[0][2] * (H[1][0] * H[2][1] - H[1][1] * H[2][0]))
        dsign = jnp.sign(detH)
        imin = jnp.where(
            eig[0] <= eig[1],
            jnp.where(eig[0] <= eig[2], 0.0, 2.0),
            jnp.where(eig[1] <= eig[2], 1.0, 2.0),
        )
        dk = []
        sinv = []
        for k in range(3):
            sk = jnp.sqrt(jnp.maximum(eig[k], 1e-30))
            dk.append(jnp.where(imin == float(k), dsign, 1.0))
            sinv.append(1.0 / sk)

        U = [[(H[j][0] * V[0][k] + H[j][1] * V[1][k] + H[j][2] * V[2][k])
              * sinv[k] for k in range(3)] for j in range(3)]
        Vb = [[_bf(V[i][k]) for k in range(3)] for i in range(3)]
        Ub = [[_bf(U[j][k]) for k in range(3)] for j in range(3)]
        Rn = [[(Vb[i][0] * dk[0] * Ub[j][0] + Vb[i][1] * dk[1] * Ub[j][1])
               + Vb[i][2] * dk[2] * Ub[j][2] for j in range(3)]
              for i in range(3)]
        Rnb = [[_bf(Rn[i][j]) for j in range(3)] for i in range(3)]
        mupb = [_bf(mu_p[0]), _bf(mu_p[1]), _bf(mu_p[2])]
        tn = [mu_q[i] - ((Rnb[i][0] * mupb[0] + Rnb[i][1] * mupb[1])
                         + Rnb[i][2] * mupb[2]) for i in range(3)]
        tnb = [_bf(tn[0]), _bf(tn[1]), _bf(tn[2])]
        tb = [_bf(t[0]), _bf(t[1]), _bf(t[2])]

        Rnew = [[(Rnb[i][0] * Rb[0][j] + Rnb[i][1] * Rb[1][j])
                 + Rnb[i][2] * Rb[2][j] for j in range(3)] for i in range(3)]
        tnew = [((Rnb[i][0] * tb[0] + Rnb[i][1] * tb[1])
                 + Rnb[i][2] * tb[2]) + tnb[i] for i in range(3)]
        return (Rnew[0][0], Rnew[0][1], Rnew[0][2],
                Rnew[1][0], Rnew[1][1], Rnew[1][2],
                Rnew[2][0], Rnew[2][1], Rnew[2][2],
                tnew[0], tnew[1], tnew[2])

    init = (R0[0][0], R0[0][1], R0[0][2],
            R0[1][0], R0[1][1], R0[1][2],
            R0[2][0], R0[2][1], R0[2][2],
            t0[0], t0[1], t0[2])
    fin = jax.lax.fori_loop(0, _ITERS, body, init)

    Rf = [[fin[0], fin[1], fin[2]], [fin[3], fin[4], fin[5]],
          [fin[6], fin[7], fin[8]]]
    tf = [fin[9], fin[10], fin[11]]
    zero = jnp.zeros((1, 1), jnp.float32)
    one_ = jnp.ones((1, 1), jnp.float32)
    row3 = jnp.concatenate([zero, zero, zero, one_], axis=1)
    for b in range(_B):
        rows = [jnp.concatenate(
            [Rf[i][0][0:1, b:b + 1], Rf[i][1][0:1, b:b + 1],
             Rf[i][2][0:1, b:b + 1], tf[i][0:1, b:b + 1]], axis=1)
            for i in range(3)]
        out_ref[b] = jnp.concatenate([rows[0], rows[1], rows[2], row3],
                                     axis=0)


def kernel(scan_pc, map_pc, T_init, params):
    mapT = map_pc.transpose(0, 2, 1)
    p2d = jnp.reshape(params.astype(jnp.float32), (1, 1))
    return pl.pallas_call(
        _icp_body,
        in_specs=[
            pl.BlockSpec((_B, _N, 3), lambda: (0, 0, 0)),
            pl.BlockSpec((_B, _M, 3), lambda: (0, 0, 0)),
            pl.BlockSpec((_B, 3, _M), lambda: (0, 0, 0)),
            pl.BlockSpec((_B, 4, 4), lambda: (0, 0, 0)),
            pl.BlockSpec((1, 1), lambda: (0, 0)),
        ],
        out_specs=pl.BlockSpec((_B, 4, 4), lambda: (0, 0, 0)),
        out_shape=jax.ShapeDtypeStruct((_B, 4, 4), jnp.float32),
    )(scan_pc, map_pc, mapT, T_init, p2d)


# speedup vs baseline: 1.8896x; 1.8896x over previous
import jax
import jax.numpy as jnp
from jax.experimental import pallas as pl
from jax.experimental.pallas import tpu as pltpu

_B, _N, _M = 8, 512, 2048
_SCALE_DIV = 1.2
_ITERS = 8
_TRIM = 5.0
_HUBER = 1.0
_SWEEPS = 5


def _bf(x):
    return x.astype(jnp.bfloat16).astype(jnp.float32)


def _icp_body(scanT_ref, mapT_ref, tinit_ref, p_ref, out_ref):
    scale = jnp.maximum(p_ref[0:1, 0:1], 0.0)

    SX = (scanT_ref[0] / _SCALE_DIV) * scale
    SY = (scanT_ref[1] / _SCALE_DIV) * scale
    SZ = (scanT_ref[2] / _SCALE_DIV) * scale
    SXB, SYB, SZB = _bf(SX), _bf(SY), _bf(SZ)

    mxs, mbs, msqs = [], [], []
    for b in range(_B):
        mx = mapT_ref[b, 0:1, :]
        my = mapT_ref[b, 1:2, :]
        mz = mapT_ref[b, 2:3, :]
        mxs.append((mx, my, mz))
        mbs.append((_bf(mx), _bf(my), _bf(mz)))
        msqs.append(mx * mx + my * my + mz * mz)
    iota = jax.lax.broadcasted_iota(jnp.int32, (_N, _M), 1).astype(jnp.float32)

    def tcol(i, j):
        return jnp.concatenate(
            [tinit_ref[b, i:i + 1, j:j + 1] for b in range(_B)], axis=1)

    R0 = [[tcol(i, j) for j in range(3)] for i in range(3)]
    t0 = [tcol(i, 3) for i in range(3)]

    def body(_, carry):
        (r00, r01, r02, r10, r11, r12, r20, r21, r22, t0_, t1_, t2_) = carry
        R = [[r00, r01, r02], [r10, r11, r12], [r20, r21, r22]]
        t = [t0_, t1_, t2_]
        Rb = [[_bf(R[i][j]) for j in range(3)] for i in range(3)]

        PX = (SXB * Rb[0][0] + SYB * Rb[0][1]) + SZB * Rb[0][2] + t[0]
        PY = (SXB * Rb[1][0] + SYB * Rb[1][1]) + SZB * Rb[1][2] + t[1]
        PZ = (SXB * Rb[2][0] + SYB * Rb[2][1]) + SZB * Rb[2][2] + t[2]
        P_SQ = PX * PX + PY * PY + PZ * PZ
        PXB, PYB, PZB = _bf(PX), _bf(PY), _bf(PZ)

        nxl, nyl, nzl = [], [], []
        for b in range(_B):
            mxb, myb, mzb = mbs[b]
            pxb = PXB[:, b:b + 1]
            pyb = PYB[:, b:b + 1]
            pzb = PZB[:, b:b + 1]

            cross = (pxb * mxb + pyb * myb) + pzb * mzb
            d2 = (P_SQ[:, b:b + 1] + msqs[b]) - 2.0 * cross
            d2min = jnp.min(d2, axis=1, keepdims=True)
            hit = d2 == d2min
            idx = jnp.min(jnp.where(hit, iota, float(_M)), axis=1,
                          keepdims=True)
            one = iota == idx

            mx, my, mz = mxs[b]
            nxl.append(jnp.sum(jnp.where(one, mx, 0.0), axis=1,
                               keepdims=True))
            nyl.append(jnp.sum(jnp.where(one, my, 0.0), axis=1,
                               keepdims=True))
            nzl.append(jnp.sum(jnp.where(one, mz, 0.0), axis=1,
                               keepdims=True))

        NX = jnp.concatenate(nxl, axis=1)
        NY = jnp.concatenate(nyl, axis=1)
        NZ = jnp.concatenate(nzl, axis=1)

        RX = PX - NX
        RY = PY - NY
        RZ = PZ - NZ
        DIST = jnp.sqrt(RX * RX + RY * RY + RZ * RZ + 1e-12)
        W_TRIM = (DIST < _TRIM).astype(jnp.float32)
        W_HUB = jnp.where(DIST > _HUBER, _HUBER / DIST, 1.0)
        W = W_TRIM * W_HUB

        def rsum(v):
            return jnp.sum(v, axis=0, keepdims=True)

        sw = rsum(W) + 1e-9
        mu_p = [rsum(W * PX) / sw, rsum(W * PY) / sw, rsum(W * PZ) / sw]
        mu_q = [rsum(W * NX) / sw, rsum(W * NY) / sw, rsum(W * NZ) / sw]
        PC = [PX - mu_p[0], PY - mu_p[1], PZ - mu_p[2]]
        QC = [NX - mu_q[0], NY - mu_q[1], NZ - mu_q[2]]
        WPCB = [_bf(W * PC[0]), _bf(W * PC[1]), _bf(W * PC[2])]
        QCB = [_bf(QC[0]), _bf(QC[1]), _bf(QC[2])]
        H = [[rsum(WPCB[i] * QCB[j]) for j in range(3)] for i in range(3)]

        def ata(i, j):
            return H[0][i] * H[0][j] + H[1][i] * H[1][j] + H[2][i] * H[2][j]

        a = [[ata(i, j) for j in range(3)] for i in range(3)]
        V = [[jnp.full((1, _B), 1.0 if i == j else 0.0, jnp.float32)
              for j in range(3)] for i in range(3)]

        for _s in range(_SWEEPS):
            for (p, q) in ((0, 1), (0, 2), (1, 2)):
                r = 3 - p - q
                app, aqq, apq = a[p][p], a[q][q], a[p][q]
                tiny = jnp.abs(apq) < 1e-37
                apq_safe = jnp.where(tiny, 1.0, apq)
                tau = (aqq - app) * 0.5 / apq_safe
                sgn = jnp.where(tau >= 0.0, 1.0, -1.0)
                tt = sgn / (jnp.abs(tau) + jnp.sqrt(1.0 + tau * tau))
                c = 1.0 / jnp.sqrt(1.0 + tt * tt)
                s = tt * c
                c = jnp.where(tiny, 1.0, c)
                s = jnp.where(tiny, 0.0, s)
                new_pp = c * c * app - 2.0 * s * c * apq + s * s * aqq
                new_qq = s * s * app + 2.0 * s * c * apq + c * c * aqq
                apr, aqr = a[p][r], a[q][r]
                new_pr = c * apr - s * aqr
                new_qr = s * apr + c * aqr
                a[p][p] = new_pp
                a[q][q] = new_qq
                a[p][q] = jnp.zeros((1, _B), jnp.float32)
                a[q][p] = a[p][q]
                a[p][r] = new_pr
                a[r][p] = new_pr
                a[q][r] = new_qr
                a[r][q] = new_qr
                for i in range(3):
                    vip, viq = V[i][p], V[i][q]
                    V[i][p] = c * vip - s * viq
                    V[i][q] = s * vip + c * viq

        eig = [a[0][0], a[1][1], a[2][2]]
        detH = (H[0][0] * (H[1][1] * H[2][2] - H[1][2] * H[2][1])
                - H[0][1] * (H[1][0] * H[2][2] - H[1][2] * H[2][0])
                + H[0][2] * (H[1][0] * H[2][1] - H[1][1] * H[2][0]))
        dsign = jnp.sign(detH)
        imin = jnp.where(
            eig[0] <= eig[1],
            jnp.where(eig[0] <= eig[2], 0.0, 2.0),
            jnp.where(eig[1] <= eig[2], 1.0, 2.0),
        )
        dk = []
        sinv = []
        for k in range(3):
            sk = jnp.sqrt(jnp.maximum(eig[k], 1e-30))
            dk.append(jnp.where(imin == float(k), dsign, 1.0))
            sinv.append(1.0 / sk)

        U = [[(H[j][0] * V[0][k] + H[j][1] * V[1][k] + H[j][2] * V[2][k])
              * sinv[k] for k in range(3)] for j in range(3)]
        Vb = [[_bf(V[i][k]) for k in range(3)] for i in range(3)]
        Ub = [[_bf(U[j][k]) for k in range(3)] for j in range(3)]
        Rn = [[(Vb[i][0] * dk[0] * Ub[j][0] + Vb[i][1] * dk[1] * Ub[j][1])
               + Vb[i][2] * dk[2] * Ub[j][2] for j in range(3)]
              for i in range(3)]
        Rnb = [[_bf(Rn[i][j]) for j in range(3)] for i in range(3)]
        mupb = [_bf(mu_p[0]), _bf(mu_p[1]), _bf(mu_p[2])]
        tn = [mu_q[i] - ((Rnb[i][0] * mupb[0] + Rnb[i][1] * mupb[1])
                         + Rnb[i][2] * mupb[2]) for i in range(3)]
        tnb = [_bf(tn[0]), _bf(tn[1]), _bf(tn[2])]
        tb = [_bf(t[0]), _bf(t[1]), _bf(t[2])]

        Rnew = [[(Rnb[i][0] * Rb[0][j] + Rnb[i][1] * Rb[1][j])
                 + Rnb[i][2] * Rb[2][j] for j in range(3)] for i in range(3)]
        tnew = [((Rnb[i][0] * tb[0] + Rnb[i][1] * tb[1])
                 + Rnb[i][2] * tb[2]) + tnb[i] for i in range(3)]
        return (Rnew[0][0], Rnew[0][1], Rnew[0][2],
                Rnew[1][0], Rnew[1][1], Rnew[1][2],
                Rnew[2][0], Rnew[2][1], Rnew[2][2],
                tnew[0], tnew[1], tnew[2])

    init = (R0[0][0], R0[0][1], R0[0][2],
            R0[1][0], R0[1][1], R0[1][2],
            R0[2][0], R0[2][1], R0[2][2],
            t0[0], t0[1], t0[2])
    fin = jax.lax.fori_loop(0, _ITERS, body, init)

    Rf = [[fin[0], fin[1], fin[2]], [fin[3], fin[4], fin[5]],
          [fin[6], fin[7], fin[8]]]
    tf = [fin[9], fin[10], fin[11]]
    zero = jnp.zeros((1, 1), jnp.float32)
    one_ = jnp.ones((1, 1), jnp.float32)
    row3 = jnp.concatenate([zero, zero, zero, one_], axis=1)
    for b in range(_B):
        rows = [jnp.concatenate(
            [Rf[i][0][0:1, b:b + 1], Rf[i][1][0:1, b:b + 1],
             Rf[i][2][0:1, b:b + 1], tf[i][0:1, b:b + 1]], axis=1)
            for i in range(3)]
        out_ref[b] = jnp.concatenate([rows[0], rows[1], rows[2], row3],
                                     axis=0)


def kernel(scan_pc, map_pc, T_init, params):
    scanT = scan_pc.transpose(2, 1, 0)
    mapT = map_pc.transpose(0, 2, 1)
    p2d = jnp.reshape(params.astype(jnp.float32), (1, 1))
    return pl.pallas_call(
        _icp_body,
        in_specs=[
            pl.BlockSpec((3, _N, _B), lambda: (0, 0, 0)),
            pl.BlockSpec((_B, 3, _M), lambda: (0, 0, 0)),
            pl.BlockSpec((_B, 4, 4), lambda: (0, 0, 0)),
            pl.BlockSpec((1, 1), lambda: (0, 0)),
        ],
        out_specs=pl.BlockSpec((_B, 4, 4), lambda: (0, 0, 0)),
        out_shape=jax.ShapeDtypeStruct((_B, 4, 4), jnp.float32),
    )(scanT, mapT, T_init, p2d)


# speedup vs baseline: 2.4974x; 1.3217x over previous
import jax
import jax.numpy as jnp
from jax.experimental import pallas as pl
from jax.experimental.pallas import tpu as pltpu

_B, _N, _M = 8, 512, 2048
_SCALE_DIV = 1.2
_ITERS = 8
_TRIM = 5.0
_HUBER = 1.0
_SWEEPS = 5


def _bf(x):
    return x.astype(jnp.bfloat16).astype(jnp.float32)


def _icp_body(scanT_ref, mapT_ref, tinit_ref, p_ref, out_ref):
    scale = jnp.maximum(p_ref[0:1, 0:1], 0.0)

    SX = (scanT_ref[0] / _SCALE_DIV) * scale
    SY = (scanT_ref[1] / _SCALE_DIV) * scale
    SZ = (scanT_ref[2] / _SCALE_DIV) * scale
    SXB, SYB, SZB = _bf(SX), _bf(SY), _bf(SZ)

    mxs, mT2bs, msqs = [], [], []
    for b in range(_B):
        mx = mapT_ref[b, 0:1, :]
        my = mapT_ref[b, 1:2, :]
        mz = mapT_ref[b, 2:3, :]
        mxs.append((mx, my, mz))
        mT2bs.append((mapT_ref[b] * -2.0).astype(jnp.bfloat16))
        msqs.append(mx * mx + my * my + mz * mz)
    iota = jax.lax.broadcasted_iota(jnp.int32, (_N, _M), 1).astype(jnp.float32)

    def tcol(i, j):
        return jnp.concatenate(
            [tinit_ref[b, i:i + 1, j:j + 1] for b in range(_B)], axis=1)

    R0 = [[tcol(i, j) for j in range(3)] for i in range(3)]
    t0 = [tcol(i, 3) for i in range(3)]

    def body(_, carry):
        (r00, r01, r02, r10, r11, r12, r20, r21, r22, t0_, t1_, t2_) = carry
        R = [[r00, r01, r02], [r10, r11, r12], [r20, r21, r22]]
        t = [t0_, t1_, t2_]
        Rb = [[_bf(R[i][j]) for j in range(3)] for i in range(3)]

        PX = (SXB * Rb[0][0] + SYB * Rb[0][1]) + SZB * Rb[0][2] + t[0]
        PY = (SXB * Rb[1][0] + SYB * Rb[1][1]) + SZB * Rb[1][2] + t[1]
        PZ = (SXB * Rb[2][0] + SYB * Rb[2][1]) + SZB * Rb[2][2] + t[2]
        P_SQ = PX * PX + PY * PY + PZ * PZ
        PXB, PYB, PZB = _bf(PX), _bf(PY), _bf(PZ)

        nxl, nyl, nzl = [], [], []
        for b in range(_B):
            ptsb = jnp.concatenate(
                [PXB[:, b:b + 1], PYB[:, b:b + 1], PZB[:, b:b + 1]],
                axis=1).astype(jnp.bfloat16)
            crossm2 = jax.lax.dot_general(
                ptsb, mT2bs[b], (((1,), (0,)), ((), ())),
                preferred_element_type=jnp.float32)
            d2 = (P_SQ[:, b:b + 1] + msqs[b]) + crossm2
            d2min = jnp.min(d2, axis=1, keepdims=True)
            hit = d2 == d2min
            idx = jnp.min(jnp.where(hit, iota, float(_M)), axis=1,
                          keepdims=True)
            one = iota == idx

            mx, my, mz = mxs[b]
            nxl.append(jnp.sum(jnp.where(one, mx, 0.0), axis=1,
                               keepdims=True))
            nyl.append(jnp.sum(jnp.where(one, my, 0.0), axis=1,
                               keepdims=True))
            nzl.append(jnp.sum(jnp.where(one, mz, 0.0), axis=1,
                               keepdims=True))

        NX = jnp.concatenate(nxl, axis=1)
        NY = jnp.concatenate(nyl, axis=1)
        NZ = jnp.concatenate(nzl, axis=1)

        RX = PX - NX
        RY = PY - NY
        RZ = PZ - NZ
        DIST = jnp.sqrt(RX * RX + RY * RY + RZ * RZ + 1e-12)
        W_TRIM = (DIST < _TRIM).astype(jnp.float32)
        W_HUB = jnp.where(DIST > _HUBER, _HUBER / DIST, 1.0)
        W = W_TRIM * W_HUB

        def rsum(v):
            return jnp.sum(v, axis=0, keepdims=True)

        sw = rsum(W) + 1e-9
        mu_p = [rsum(W * PX) / sw, rsum(W * PY) / sw, rsum(W * PZ) / sw]
        mu_q = [rsum(W * NX) / sw, rsum(W * NY) / sw, rsum(W * NZ) / sw]
        PC = [PX - mu_p[0], PY - mu_p[1], PZ - mu_p[2]]
        QC = [NX - mu_q[0], NY - mu_q[1], NZ - mu_q[2]]
        WPCB = [_bf(W * PC[0]), _bf(W * PC[1]), _bf(W * PC[2])]
        QCB = [_bf(QC[0]), _bf(QC[1]), _bf(QC[2])]
        H = [[rsum(WPCB[i] * QCB[j]) for j in range(3)] for i in range(3)]

        def ata(i, j):
            return H[0][i] * H[0][j] + H[1][i] * H[1][j] + H[2][i] * H[2][j]

        a = [[ata(i, j) for j in range(3)] for i in range(3)]
        V = [[jnp.full((1, _B), 1.0 if i == j else 0.0, jnp.float32)
              for j in range(3)] for i in range(3)]

        for _s in range(_SWEEPS):
            for (p, q) in ((0, 1), (0, 2), (1, 2)):
                r = 3 - p - q
                app, aqq, apq = a[p][p], a[q][q], a[p][q]
                tiny = jnp.abs(apq) < 1e-37
                apq_safe = jnp.where(tiny, 1.0, apq)
                tau = (aqq - app) * 0.5 / apq_safe
                sgn = jnp.where(tau >= 0.0, 1.0, -1.0)
                tt = sgn / (jnp.abs(tau) + jnp.sqrt(1.0 + tau * tau))
                c = 1.0 / jnp.sqrt(1.0 + tt * tt)
                s = tt * c
                c = jnp.where(tiny, 1.0, c)
                s = jnp.where(tiny, 0.0, s)
                new_pp = c * c * app - 2.0 * s * c * apq + s * s * aqq
                new_qq = s * s * app + 2.0 * s * c * apq + c * c * aqq
                apr, aqr = a[p][r], a[q][r]
                new_pr = c * apr - s * aqr
                new_qr = s * apr + c * aqr
                a[p][p] = new_pp
                a[q][q] = new_qq
                a[p][q] = jnp.zeros((1, _B), jnp.float32)
                a[q][p] = a[p][q]
                a[p][r] = new_pr
                a[r][p] = new_pr
                a[q][r] = new_qr
                a[r][q] = new_qr
                for i in range(3):
                    vip, viq = V[i][p], V[i][q]
                    V[i][p] = c * vip - s * viq
                    V[i][q] = s * vip + c * viq

        eig = [a[0][0], a[1][1], a[2][2]]
        detH = (H[0][0] * (H[1][1] * H[2][2] - H[1][2] * H[2][1])
                - H[0][1] * (H[1][0] * H[2][2] - H[1][2] * H[2][0])
                + H[0][2] * (H[1][0] * H[2][1] - H[1][1] * H[2][0]))
        dsign = jnp.sign(detH)
        imin = jnp.where(
            eig[0] <= eig[1],
            jnp.where(eig[0] <= eig[2], 0.0, 2.0),
            jnp.where(eig[1] <= eig[2], 1.0, 2.0),
        )
        dk = []
        sinv = []
        for k in range(3):
            sk = jnp.sqrt(jnp.maximum(eig[k], 1e-30))
            dk.append(jnp.where(imin == float(k), dsign, 1.0))
            sinv.append(1.0 / sk)

        U = [[(H[j][0] * V[0][k] + H[j][1] * V[1][k] + H[j][2] * V[2][k])
              * sinv[k] for k in range(3)] for j in range(3)]
        Vb = [[_bf(V[i][k]) for k in range(3)] for i in range(3)]
        Ub = [[_bf(U[j][k]) for k in range(3)] for j in range(3)]
        Rn = [[(Vb[i][0] * dk[0] * Ub[j][0] + Vb[i][1] * dk[1] * Ub[j][1])
               + Vb[i][2] * dk[2] * Ub[j][2] for j in range(3)]
              for i in range(3)]
        Rnb = [[_bf(Rn[i][j]) for j in range(3)] for i in range(3)]
        mupb = [_bf(mu_p[0]), _bf(mu_p[1]), _bf(mu_p[2])]
        tn = [mu_q[i] - ((Rnb[i][0] * mupb[0] + Rnb[i][1] * mupb[1])
                         + Rnb[i][2] * mupb[2]) for i in range(3)]
        tnb = [_bf(tn[0]), _bf(tn[1]), _bf(tn[2])]
        tb = [_bf(t[0]), _bf(t[1]), _bf(t[2])]

        Rnew = [[(Rnb[i][0] * Rb[0][j] + Rnb[i][1] * Rb[1][j])
                 + Rnb[i][2] * Rb[2][j] for j in range(3)] for i in range(3)]
        tnew = [((Rnb[i][0] * tb[0] + Rnb[i][1] * tb[1])
                 + Rnb[i][2] * tb[2]) + tnb[i] for i in range(3)]
        return (Rnew[0][0], Rnew[0][1], Rnew[0][2],
                Rnew[1][0], Rnew[1][1], Rnew[1][2],
                Rnew[2][0], Rnew[2][1], Rnew[2][2],
                tnew[0], tnew[1], tnew[2])

    init = (R0[0][0], R0[0][1], R0[0][2],
            R0[1][0], R0[1][1], R0[1][2],
            R0[2][0], R0[2][1], R0[2][2],
            t0[0], t0[1], t0[2])
    fin = jax.lax.fori_loop(0, _ITERS, body, init)

    Rf = [[fin[0], fin[1], fin[2]], [fin[3], fin[4], fin[5]],
          [fin[6], fin[7], fin[8]]]
    tf = [fin[9], fin[10], fin[11]]
    zero = jnp.zeros((1, 1), jnp.float32)
    one_ = jnp.ones((1, 1), jnp.float32)
    row3 = jnp.concatenate([zero, zero, zero, one_], axis=1)
    for b in range(_B):
        rows = [jnp.concatenate(
            [Rf[i][0][0:1, b:b + 1], Rf[i][1][0:1, b:b + 1],
             Rf[i][2][0:1, b:b + 1], tf[i][0:1, b:b + 1]], axis=1)
            for i in range(3)]
        out_ref[b] = jnp.concatenate([rows[0], rows[1], rows[2], row3],
                                     axis=0)


def kernel(scan_pc, map_pc, T_init, params):
    scanT = scan_pc.transpose(2, 1, 0)
    mapT = map_pc.transpose(0, 2, 1)
    p2d = jnp.reshape(params.astype(jnp.float32), (1, 1))
    return pl.pallas_call(
        _icp_body,
        in_specs=[
            pl.BlockSpec((3, _N, _B), lambda: (0, 0, 0)),
            pl.BlockSpec((_B, 3, _M), lambda: (0, 0, 0)),
            pl.BlockSpec((_B, 4, 4), lambda: (0, 0, 0)),
            pl.BlockSpec((1, 1), lambda: (0, 0)),
        ],
        out_specs=pl.BlockSpec((_B, 4, 4), lambda: (0, 0, 0)),
        out_shape=jax.ShapeDtypeStruct((_B, 4, 4), jnp.float32),
    )(scanT, mapT, T_init, p2d)


# speedup vs baseline: 3.7219x; 1.4903x over previous
import jax
import jax.numpy as jnp
from jax.experimental import pallas as pl
from jax.experimental.pallas import tpu as pltpu

_B, _N, _M = 8, 512, 2048
_SCALE_DIV = 1.2
_ITERS = 8
_TRIM = 5.0
_HUBER = 1.0
_SWEEPS = 5


def _bf(x):
    return x.astype(jnp.bfloat16).astype(jnp.float32)


def _icp_body(scanT_ref, mapT_ref, tinit_ref, p_ref, out_ref):
    scale = jnp.maximum(p_ref[0:1, 0:1], 0.0)

    SX = (scanT_ref[0] / _SCALE_DIV) * scale
    SY = (scanT_ref[1] / _SCALE_DIV) * scale
    SZ = (scanT_ref[2] / _SCALE_DIV) * scale
    SXB, SYB, SZB = _bf(SX), _bf(SY), _bf(SZ)

    mxs, mT2bs, msqs = [], [], []
    for b in range(_B):
        mx = mapT_ref[b, 0:1, :]
        my = mapT_ref[b, 1:2, :]
        mz = mapT_ref[b, 2:3, :]
        mxs.append((mx.reshape(8, _M // 8), my.reshape(8, _M // 8),
                    mz.reshape(8, _M // 8)))
        mT2bs.append((mapT_ref[b] * -2.0).astype(jnp.bfloat16))
        msqs.append(mx * mx + my * my + mz * mz)
    iota = jax.lax.broadcasted_iota(jnp.int32, (_N, _M), 1).astype(jnp.float32)

    def tcol(i, j):
        return jnp.concatenate(
            [tinit_ref[b, i:i + 1, j:j + 1] for b in range(_B)], axis=1)

    R0 = [[tcol(i, j) for j in range(3)] for i in range(3)]
    t0 = [tcol(i, 3) for i in range(3)]

    def body(_, carry):
        (r00, r01, r02, r10, r11, r12, r20, r21, r22, t0_, t1_, t2_) = carry
        R = [[r00, r01, r02], [r10, r11, r12], [r20, r21, r22]]
        t = [t0_, t1_, t2_]
        Rb = [[_bf(R[i][j]) for j in range(3)] for i in range(3)]

        PX = (SXB * Rb[0][0] + SYB * Rb[0][1]) + SZB * Rb[0][2] + t[0]
        PY = (SXB * Rb[1][0] + SYB * Rb[1][1]) + SZB * Rb[1][2] + t[1]
        PZ = (SXB * Rb[2][0] + SYB * Rb[2][1]) + SZB * Rb[2][2] + t[2]
        P_SQ = PX * PX + PY * PY + PZ * PZ
        PXB, PYB, PZB = _bf(PX), _bf(PY), _bf(PZ)

        nxl, nyl, nzl = [], [], []
        for b in range(_B):
            ptsb = jnp.concatenate(
                [PXB[:, b:b + 1], PYB[:, b:b + 1], PZB[:, b:b + 1]],
                axis=1).astype(jnp.bfloat16)
            crossm2 = jax.lax.dot_general(
                ptsb, mT2bs[b], (((1,), (0,)), ((), ())),
                preferred_element_type=jnp.float32)
            d2 = (P_SQ[:, b:b + 1] + msqs[b]) + crossm2
            d2min = jnp.min(d2, axis=1, keepdims=True)
            hit = d2 == d2min
            idx = jnp.min(jnp.where(hit, iota, float(_M)), axis=1,
                          keepdims=True)

            it = idx.astype(jnp.int32)
            tidx = jnp.broadcast_to(
                jax.lax.shift_right_logical(it, 8), (_N, _M // 8))
            lidx = jax.lax.bitwise_and(it, 255)
            lmask = jax.lax.broadcasted_iota(
                jnp.int32, (_N, _M // 8), 1) == lidx
            mx, my, mz = mxs[b]
            nxl.append(jnp.sum(jnp.where(
                lmask, jnp.take_along_axis(mx, tidx, axis=0), 0.0),
                axis=1, keepdims=True))
            nyl.append(jnp.sum(jnp.where(
                lmask, jnp.take_along_axis(my, tidx, axis=0), 0.0),
                axis=1, keepdims=True))
            nzl.append(jnp.sum(jnp.where(
                lmask, jnp.take_along_axis(mz, tidx, axis=0), 0.0),
                axis=1, keepdims=True))

        NX = jnp.concatenate(nxl, axis=1)
        NY = jnp.concatenate(nyl, axis=1)
        NZ = jnp.concatenate(nzl, axis=1)

        RX = PX - NX
        RY = PY - NY
        RZ = PZ - NZ
        DIST = jnp.sqrt(RX * RX + RY * RY + RZ * RZ + 1e-12)
        W_TRIM = (DIST < _TRIM).astype(jnp.float32)
        W_HUB = jnp.where(DIST > _HUBER, _HUBER / DIST, 1.0)
        W = W_TRIM * W_HUB

        def rsum(v):
            return jnp.sum(v, axis=0, keepdims=True)

        sw = rsum(W) + 1e-9
        mu_p = [rsum(W * PX) / sw, rsum(W * PY) / sw, rsum(W * PZ) / sw]
        mu_q = [rsum(W * NX) / sw, rsum(W * NY) / sw, rsum(W * NZ) / sw]
        PC = [PX - mu_p[0], PY - mu_p[1], PZ - mu_p[2]]
        QC = [NX - mu_q[0], NY - mu_q[1], NZ - mu_q[2]]
        WPCB = [_bf(W * PC[0]), _bf(W * PC[1]), _bf(W * PC[2])]
        QCB = [_bf(QC[0]), _bf(QC[1]), _bf(QC[2])]
        H = [[rsum(WPCB[i] * QCB[j]) for j in range(3)] for i in range(3)]

        def ata(i, j):
            return H[0][i] * H[0][j] + H[1][i] * H[1][j] + H[2][i] * H[2][j]

        a = [[ata(i, j) for j in range(3)] for i in range(3)]
        V = [[jnp.full((1, _B), 1.0 if i == j else 0.0, jnp.float32)
              for j in range(3)] for i in range(3)]

        for _s in range(_SWEEPS):
            for (p, q) in ((0, 1), (0, 2), (1, 2)):
                r = 3 - p - q
                app, aqq, apq = a[p][p], a[q][q], a[p][q]
                tiny = jnp.abs(apq) < 1e-37
                apq_safe = jnp.where(tiny, 1.0, apq)
                tau = (aqq - app) * 0.5 / apq_safe
                sgn = jnp.where(tau >= 0.0, 1.0, -1.0)
                tt = sgn / (jnp.abs(tau) + jnp.sqrt(1.0 + tau * tau))
                c = 1.0 / jnp.sqrt(1.0 + tt * tt)
                s = tt * c
                c = jnp.where(tiny, 1.0, c)
                s = jnp.where(tiny, 0.0, s)
                new_pp = c * c * app - 2.0 * s * c * apq + s * s * aqq
                new_qq = s * s * app + 2.0 * s * c * apq + c * c * aqq
                apr, aqr = a[p][r], a[q][r]
                new_pr = c * apr - s * aqr
                new_qr = s * apr + c * aqr
                a[p][p] = new_pp
                a[q][q] = new_qq
                a[p][q] = jnp.zeros((1, _B), jnp.float32)
                a[q][p] = a[p][q]
                a[p][r] = new_pr
                a[r][p] = new_pr
                a[q][r] = new_qr
                a[r][q] = new_qr
                for i in range(3):
                    vip, viq = V[i][p], V[i][q]
                    V[i][p] = c * vip - s * viq
                    V[i][q] = s * vip + c * viq

        eig = [a[0][0], a[1][1], a[2][2]]
        detH = (H[0][0] * (H[1][1] * H[2][2] - H[1][2] * H[2][1])
                - H[0][1] * (H[1][0] * H[2][2] - H[1][2] * H[2][0])
                + H[0][2] * (H[1][0] * H[2][1] - H[1][1] * H[2][0]))
        dsign = jnp.sign(detH)
        imin = jnp.where(
            eig[0] <= eig[1],
            jnp.where(eig[0] <= eig[2], 0.0, 2.0),
            jnp.where(eig[1] <= eig[2], 1.0, 2.0),
        )
        dk = []
        sinv = []
        for k in range(3):
            sk = jnp.sqrt(jnp.maximum(eig[k], 1e-30))
            dk.append(jnp.where(imin == float(k), dsign, 1.0))
            sinv.append(1.0 / sk)

        U = [[(H[j][0] * V[0][k] + H[j][1] * V[1][k] + H[j][2] * V[2][k])
              * sinv[k] for k in range(3)] for j in range(3)]
        Vb = [[_bf(V[i][k]) for k in range(3)] for i in range(3)]
        Ub = [[_bf(U[j][k]) for k in range(3)] for j in range(3)]
        Rn = [[(Vb[i][0] * dk[0] * Ub[j][0] + Vb[i][1] * dk[1] * Ub[j][1])
               + Vb[i][2] * dk[2] * Ub[j][2] for j in range(3)]
              for i in range(3)]
        Rnb = [[_bf(Rn[i][j]) for j in range(3)] for i in range(3)]
        mupb = [_bf(mu_p[0]), _bf(mu_p[1]), _bf(mu_p[2])]
        tn = [mu_q[i] - ((Rnb[i][0] * mupb[0] + Rnb[i][1] * mupb[1])
                         + Rnb[i][2] * mupb[2]) for i in range(3)]
        tnb = [_bf(tn[0]), _bf(tn[1]), _bf(tn[2])]
        tb = [_bf(t[0]), _bf(t[1]), _bf(t[2])]

        Rnew = [[(Rnb[i][0] * Rb[0][j] + Rnb[i][1] * Rb[1][j])
                 + Rnb[i][2] * Rb[2][j] for j in range(3)] for i in range(3)]
        tnew = [((Rnb[i][0] * tb[0] + Rnb[i][1] * tb[1])
                 + Rnb[i][2] * tb[2]) + tnb[i] for i in range(3)]
        return (Rnew[0][0], Rnew[0][1], Rnew[0][2],
                Rnew[1][0], Rnew[1][1], Rnew[1][2],
                Rnew[2][0], Rnew[2][1], Rnew[2][2],
                tnew[0], tnew[1], tnew[2])

    init = (R0[0][0], R0[0][1], R0[0][2],
            R0[1][0], R0[1][1], R0[1][2],
            R0[2][0], R0[2][1], R0[2][2],
            t0[0], t0[1], t0[2])
    fin = jax.lax.fori_loop(0, _ITERS, body, init)

    Rf = [[fin[0], fin[1], fin[2]], [fin[3], fin[4], fin[5]],
          [fin[6], fin[7], fin[8]]]
    tf = [fin[9], fin[10], fin[11]]
    zero = jnp.zeros((1, 1), jnp.float32)
    one_ = jnp.ones((1, 1), jnp.float32)
    row3 = jnp.concatenate([zero, zero, zero, one_], axis=1)
    for b in range(_B):
        rows = [jnp.concatenate(
            [Rf[i][0][0:1, b:b + 1], Rf[i][1][0:1, b:b + 1],
             Rf[i][2][0:1, b:b + 1], tf[i][0:1, b:b + 1]], axis=1)
            for i in range(3)]
        out_ref[b] = jnp.concatenate([rows[0], rows[1], rows[2], row3],
                                     axis=0)


def kernel(scan_pc, map_pc, T_init, params):
    scanT = scan_pc.transpose(2, 1, 0)
    mapT = map_pc.transpose(0, 2, 1)
    p2d = jnp.reshape(params.astype(jnp.float32), (1, 1))
    return pl.pallas_call(
        _icp_body,
        in_specs=[
            pl.BlockSpec((3, _N, _B), lambda: (0, 0, 0)),
            pl.BlockSpec((_B, 3, _M), lambda: (0, 0, 0)),
            pl.BlockSpec((_B, 4, 4), lambda: (0, 0, 0)),
            pl.BlockSpec((1, 1), lambda: (0, 0)),
        ],
        out_specs=pl.BlockSpec((_B, 4, 4), lambda: (0, 0, 0)),
        out_shape=jax.ShapeDtypeStruct((_B, 4, 4), jnp.float32),
    )(scanT, mapT, T_init, p2d)


# speedup vs baseline: 4.0981x; 1.1011x over previous
import jax
import jax.numpy as jnp
from jax.experimental import pallas as pl
from jax.experimental.pallas import tpu as pltpu

_B, _N, _M = 8, 512, 2048
_SCALE_DIV = 1.2
_ITERS = 8
_TRIM = 5.0
_HUBER = 1.0
_SWEEPS = 5


def _bf(x):
    return x.astype(jnp.bfloat16).astype(jnp.float32)


def _split3(x):
    u = jax.lax.bitcast_convert_type(x, jnp.uint32)
    a = jax.lax.bitcast_convert_type(u & jnp.uint32(0xFFFF0000), jnp.float32)
    r = x - a
    ur = jax.lax.bitcast_convert_type(r, jnp.uint32)
    b = jax.lax.bitcast_convert_type(ur & jnp.uint32(0xFFFF0000), jnp.float32)
    return a, b, r - b


def _icp_body(scanT_ref, mapT_ref, tinit_ref, p_ref, out_ref):
    scale = jnp.maximum(p_ref[0:1, 0:1], 0.0)

    SX = (scanT_ref[0] / _SCALE_DIV) * scale
    SY = (scanT_ref[1] / _SCALE_DIV) * scale
    SZ = (scanT_ref[2] / _SCALE_DIV) * scale
    SXB, SYB, SZB = _bf(SX), _bf(SY), _bf(SZ)

    ones_row = jnp.ones((1, _M), jnp.float32)
    mxs, mT2bs = [], []
    for b in range(_B):
        mx = mapT_ref[b, 0:1, :]
        my = mapT_ref[b, 1:2, :]
        mz = mapT_ref[b, 2:3, :]
        mxs.append((mx.reshape(8, _M // 8), my.reshape(8, _M // 8),
                    mz.reshape(8, _M // 8)))
        msq = mx * mx + my * my + mz * mz
        q1, q2, q3 = _split3(msq)
        mT2bs.append(jnp.concatenate(
            [mapT_ref[b] * -2.0, ones_row, ones_row, ones_row, q1, q2, q3],
            axis=0).astype(jnp.bfloat16))
    iota = jax.lax.broadcasted_iota(jnp.int32, (_N, _M), 1).astype(jnp.float32)

    def tcol(i, j):
        return jnp.concatenate(
            [tinit_ref[b, i:i + 1, j:j + 1] for b in range(_B)], axis=1)

    R0 = [[tcol(i, j) for j in range(3)] for i in range(3)]
    t0 = [tcol(i, 3) for i in range(3)]

    def body(_, carry):
        (r00, r01, r02, r10, r11, r12, r20, r21, r22, t0_, t1_, t2_) = carry
        R = [[r00, r01, r02], [r10, r11, r12], [r20, r21, r22]]
        t = [t0_, t1_, t2_]
        Rb = [[_bf(R[i][j]) for j in range(3)] for i in range(3)]

        PX = (SXB * Rb[0][0] + SYB * Rb[0][1]) + SZB * Rb[0][2] + t[0]
        PY = (SXB * Rb[1][0] + SYB * Rb[1][1]) + SZB * Rb[1][2] + t[1]
        PZ = (SXB * Rb[2][0] + SYB * Rb[2][1]) + SZB * Rb[2][2] + t[2]
        P_SQ = PX * PX + PY * PY + PZ * PZ
        PXB, PYB, PZB = _bf(PX), _bf(PY), _bf(PZ)

        ones_col = jnp.ones((_N, 1), jnp.float32)
        nxl, nyl, nzl = [], [], []
        for b in range(_B):
            s1, s2, s3 = _split3(P_SQ[:, b:b + 1])
            ptsb = jnp.concatenate(
                [PXB[:, b:b + 1], PYB[:, b:b + 1], PZB[:, b:b + 1],
                 s1, s2, s3, ones_col, ones_col, ones_col],
                axis=1).astype(jnp.bfloat16)
            d2 = jax.lax.dot_general(
                ptsb, mT2bs[b], (((1,), (0,)), ((), ())),
                preferred_element_type=jnp.float32)
            d2min = jnp.min(d2, axis=1, keepdims=True)
            hit = d2 == d2min
            idx = jnp.min(jnp.where(hit, iota, float(_M)), axis=1,
                          keepdims=True)

            it = idx.astype(jnp.int32)
            tidx = jnp.broadcast_to(
                jax.lax.shift_right_logical(it, 8), (_N, _M // 8))
            lidx = jax.lax.bitwise_and(it, 255)
            lmask = jax.lax.broadcasted_iota(
                jnp.int32, (_N, _M // 8), 1) == lidx
            mx, my, mz = mxs[b]
            nxl.append(jnp.sum(jnp.where(
                lmask, jnp.take_along_axis(mx, tidx, axis=0), 0.0),
                axis=1, keepdims=True))
            nyl.append(jnp.sum(jnp.where(
                lmask, jnp.take_along_axis(my, tidx, axis=0), 0.0),
                axis=1, keepdims=True))
            nzl.append(jnp.sum(jnp.where(
                lmask, jnp.take_along_axis(mz, tidx, axis=0), 0.0),
                axis=1, keepdims=True))

        NX = jnp.concatenate(nxl, axis=1)
        NY = jnp.concatenate(nyl, axis=1)
        NZ = jnp.concatenate(nzl, axis=1)

        RX = PX - NX
        RY = PY - NY
        RZ = PZ - NZ
        DIST = jnp.sqrt(RX * RX + RY * RY + RZ * RZ + 1e-12)
        W_TRIM = (DIST < _TRIM).astype(jnp.float32)
        W_HUB = jnp.where(DIST > _HUBER, _HUBER / DIST, 1.0)
        W = W_TRIM * W_HUB

        def rsum(v):
            return jnp.sum(v, axis=0, keepdims=True)

        sw = rsum(W) + 1e-9
        mu_p = [rsum(W * PX) / sw, rsum(W * PY) / sw, rsum(W * PZ) / sw]
        mu_q = [rsum(W * NX) / sw, rsum(W * NY) / sw, rsum(W * NZ) / sw]
        PC = [PX - mu_p[0], PY - mu_p[1], PZ - mu_p[2]]
        QC = [NX - mu_q[0], NY - mu_q[1], NZ - mu_q[2]]
        WPCB = [_bf(W * PC[0]), _bf(W * PC[1]), _bf(W * PC[2])]
        QCB = [_bf(QC[0]), _bf(QC[1]), _bf(QC[2])]
        H = [[rsum(WPCB[i] * QCB[j]) for j in range(3)] for i in range(3)]

        def ata(i, j):
            return H[0][i] * H[0][j] + H[1][i] * H[1][j] + H[2][i] * H[2][j]

        a = [[ata(i, j) for j in range(3)] for i in range(3)]
        V = [[jnp.full((1, _B), 1.0 if i == j else 0.0, jnp.float32)
              for j in range(3)] for i in range(3)]

        for _s in range(_SWEEPS):
            for (p, q) in ((0, 1), (0, 2), (1, 2)):
                r = 3 - p - q
                app, aqq, apq = a[p][p], a[q][q], a[p][q]
                tiny = jnp.abs(apq) < 1e-37
                apq_safe = jnp.where(tiny, 1.0, apq)
                tau = (aqq - app) * 0.5 / apq_safe
                sgn = jnp.where(tau >= 0.0, 1.0, -1.0)
                tt = sgn / (jnp.abs(tau) + jnp.sqrt(1.0 + tau * tau))
                c = 1.0 / jnp.sqrt(1.0 + tt * tt)
                s = tt * c
                c = jnp.where(tiny, 1.0, c)
                s = jnp.where(tiny, 0.0, s)
                new_pp = c * c * app - 2.0 * s * c * apq + s * s * aqq
                new_qq = s * s * app + 2.0 * s * c * apq + c * c * aqq
                apr, aqr = a[p][r], a[q][r]
                new_pr = c * apr - s * aqr
                new_qr = s * apr + c * aqr
                a[p][p] = new_pp
                a[q][q] = new_qq
                a[p][q] = jnp.zeros((1, _B), jnp.float32)
                a[q][p] = a[p][q]
                a[p][r] = new_pr
                a[r][p] = new_pr
                a[q][r] = new_qr
                a[r][q] = new_qr
                for i in range(3):
                    vip, viq = V[i][p], V[i][q]
                    V[i][p] = c * vip - s * viq
                    V[i][q] = s * vip + c * viq

        eig = [a[0][0], a[1][1], a[2][2]]
        detH = (H[0][0] * (H[1][1] * H[2][2] - H[1][2] * H[2][1])
                - H[0][1] * (H[1][0] * H[2][2] - H[1][2] * H[2][0])
                + H[0][2] * (H[1][0] * H[2][1] - H[1][1] * H[2][0]))
        dsign = jnp.sign(detH)
        imin = jnp.where(
            eig[0] <= eig[1],
            jnp.where(eig[0] <= eig[2], 0.0, 2.0),
            jnp.where(eig[1] <= eig[2], 1.0, 2.0),
        )
        dk = []
        sinv = []
        for k in range(3):
            sk = jnp.sqrt(jnp.maximum(eig[k], 1e-30))
            dk.append(jnp.where(imin == float(k), dsign, 1.0))
            sinv.append(1.0 / sk)

        U = [[(H[j][0] * V[0][k] + H[j][1] * V[1][k] + H[j][2] * V[2][k])
              * sinv[k] for k in range(3)] for j in range(3)]
        Vb = [[_bf(V[i][k]) for k in range(3)] for i in range(3)]
        Ub = [[_bf(U[j][k]) for k in range(3)] for j in range(3)]
        Rn = [[(Vb[i][0] * dk[0] * Ub[j][0] + Vb[i][1] * dk[1] * Ub[j][1])
               + Vb[i][2] * dk[2] * Ub[j][2] for j in range(3)]
              for i in range(3)]
        Rnb = [[_bf(Rn[i][j]) for j in range(3)] for i in range(3)]
        mupb = [_bf(mu_p[0]), _bf(mu_p[1]), _bf(mu_p[2])]
        tn = [mu_q[i] - ((Rnb[i][0] * mupb[0] + Rnb[i][1] * mupb[1])
                         + Rnb[i][2] * mupb[2]) for i in range(3)]
        tnb = [_bf(tn[0]), _bf(tn[1]), _bf(tn[2])]
        tb = [_bf(t[0]), _bf(t[1]), _bf(t[2])]

        Rnew = [[(Rnb[i][0] * Rb[0][j] + Rnb[i][1] * Rb[1][j])
                 + Rnb[i][2] * Rb[2][j] for j in range(3)] for i in range(3)]
        tnew = [((Rnb[i][0] * tb[0] + Rnb[i][1] * tb[1])
                 + Rnb[i][2] * tb[2]) + tnb[i] for i in range(3)]
        return (Rnew[0][0], Rnew[0][1], Rnew[0][2],
                Rnew[1][0], Rnew[1][1], Rnew[1][2],
                Rnew[2][0], Rnew[2][1], Rnew[2][2],
                tnew[0], tnew[1], tnew[2])

    init = (R0[0][0], R0[0][1], R0[0][2],
            R0[1][0], R0[1][1], R0[1][2],
            R0[2][0], R0[2][1], R0[2][2],
            t0[0], t0[1], t0[2])
    fin = jax.lax.fori_loop(0, _ITERS, body, init)

    Rf = [[fin[0], fin[1], fin[2]], [fin[3], fin[4], fin[5]],
          [fin[6], fin[7], fin[8]]]
    tf = [fin[9], fin[10], fin[11]]
    zero = jnp.zeros((1, 1), jnp.float32)
    one_ = jnp.ones((1, 1), jnp.float32)
    row3 = jnp.concatenate([zero, zero, zero, one_], axis=1)
    for b in range(_B):
        rows = [jnp.concatenate(
            [Rf[i][0][0:1, b:b + 1], Rf[i][1][0:1, b:b + 1],
             Rf[i][2][0:1, b:b + 1], tf[i][0:1, b:b + 1]], axis=1)
            for i in range(3)]
        out_ref[b] = jnp.concatenate([rows[0], rows[1], rows[2], row3],
                                     axis=0)


def kernel(scan_pc, map_pc, T_init, params):
    scanT = scan_pc.transpose(2, 1, 0)
    mapT = map_pc.transpose(0, 2, 1)
    p2d = jnp.reshape(params.astype(jnp.float32), (1, 1))
    return pl.pallas_call(
        _icp_body,
        in_specs=[
            pl.BlockSpec((3, _N, _B), lambda: (0, 0, 0)),
            pl.BlockSpec((_B, 3, _M), lambda: (0, 0, 0)),
            pl.BlockSpec((_B, 4, 4), lambda: (0, 0, 0)),
            pl.BlockSpec((1, 1), lambda: (0, 0)),
        ],
        out_specs=pl.BlockSpec((_B, 4, 4), lambda: (0, 0, 0)),
        out_shape=jax.ShapeDtypeStruct((_B, 4, 4), jnp.float32),
    )(scanT, mapT, T_init, p2d)
